# baseline (speedup 1.0000x reference)
import functools

import jax
import jax.numpy as jnp
from jax import lax
from jax.experimental import pallas as pl
from jax.experimental.pallas import tpu as pltpu
from jax.experimental.pallas import tpu_sc as plsc

_D = 65536
_d = 1024
_NC = 2
_NS = 16
_NW = _NC * _NS

_SC_D = 65536
_TC_D = _D - _SC_D
_RW = _SC_D // _NW
_CH = 32
_NCH = _RW // _CH
_CHW = _CH * _d
_JU = 4

_TC_BLK = 4096


def _sc_body(z_hbm, p_hbm, t_hbm, out_hbm, z_v, pa_v, pb_v, o_v, t_v, s_v, sem_a, sem_b):
    wid = lax.axis_index("s") * _NC + lax.axis_index("c")
    out0 = wid * _RW
    row0 = _TC_D + out0

    pltpu.sync_copy(z_hbm, z_v)
    pltpu.sync_copy(t_hbm.at[pl.ds(row0, _RW)], t_v)

    pltpu.async_copy(p_hbm.at[pl.ds(row0, _CH), :], pa_v, sem_a)
    pltpu.async_copy(p_hbm.at[pl.ds(row0 + _CH, _CH), :], pb_v, sem_b)

    def wait_chunk(buf, sem):
        pltpu.make_async_copy(p_hbm.at[pl.ds(0, _CH), :], buf, sem).wait()

    def compute_chunk(p_v, g):
        for gi in range(_CH // 16):

            @plsc.parallel_loop(
                0, _d // 16, unroll=_JU,
                carry=tuple(jnp.zeros((16,), jnp.float32) for _ in range(16)),
            )
            def accs(j, accs):
                accs = list(accs)
                zj = z_v[pl.ds(j * 16, 16)]
                for r in range(16):
                    pv = p_v[gi * 16 + r, pl.ds(j * 16, 16)]
                    accs[r] = accs[r] + pv * zj
                return tuple(accs)

            off = g * _CH + gi * 16
            for r in range(16):
                base = 32 * r
                a = accs[r]
                s_v[pl.ds(base, 16)] = a
                a = a + s_v[pl.ds(base + 8, 16)]
                s_v[pl.ds(base, 16)] = a
                a = a + s_v[pl.ds(base + 4, 16)]
                s_v[pl.ds(base, 16)] = a
                a = a + s_v[pl.ds(base + 2, 16)]
                s_v[pl.ds(base, 16)] = a
                a = a + s_v[pl.ds(base + 1, 16)]
                o_v[pl.ds(off + r, 16)] = a

    def pair_body(k, _):
        g_a = 2 * k
        wait_chunk(pa_v, sem_a)
        compute_chunk(pa_v, g_a)

        @pl.when(g_a + 2 < _NCH)
        def _():
            pltpu.async_copy(
                p_hbm.at[pl.ds(row0 + (g_a + 2) * _CH, _CH), :], pa_v, sem_a)

        wait_chunk(pb_v, sem_b)
        compute_chunk(pb_v, g_a + 1)

        @pl.when(g_a + 3 < _NCH)
        def _():
            pltpu.async_copy(
                p_hbm.at[pl.ds(row0 + (g_a + 3) * _CH, _CH), :], pb_v, sem_b)

        return 0

    lax.fori_loop(0, _NCH // 2, pair_body, 0)

    def theta_body(k, _):
        sl = pl.ds(k * 16, 16)
        o_v[sl] = o_v[sl] + t_v[sl]
        return 0

    lax.fori_loop(0, _RW // 16, theta_body, 0)

    pltpu.sync_copy(o_v.at[pl.ds(0, _RW)], out_hbm.at[pl.ds(out0, _RW)])


_sc_call = functools.partial(
    pl.kernel,
    out_type=jax.ShapeDtypeStruct((_SC_D,), jnp.float32),
    mesh=plsc.VectorSubcoreMesh(core_axis_name="c", subcore_axis_name="s"),
    scratch_types=[
        pltpu.VMEM((_d,), jnp.float32),
        pltpu.VMEM((_CH, _d), jnp.float32),
        pltpu.VMEM((_CH, _d), jnp.float32),
        pltpu.VMEM((_RW + 16,), jnp.float32),
        pltpu.VMEM((_RW,), jnp.float32),
        pltpu.VMEM((512,), jnp.float32),
        pltpu.SemaphoreType.DMA,
        pltpu.SemaphoreType.DMA,
    ],
)(_sc_body)


def _tc_body(p_ref, z_ref, t_ref, o_ref):
    o_ref[...] = t_ref[...] + jnp.sum(p_ref[...] * z_ref[...], axis=1)


def kernel(z, P, theta_base):
    if _TC_D == 0:
        return _sc_call(z, P, theta_base)
    zb = z.reshape(1, _d)
    out_sc = _sc_call(z, P, theta_base)
    out_tc = pl.pallas_call(
        _tc_body,
        grid=(_TC_D // _TC_BLK,),
        in_specs=[
            pl.BlockSpec((_TC_BLK, _d), lambda i: (i, 0)),
            pl.BlockSpec((1, _d), lambda i: (0, 0)),
            pl.BlockSpec((_TC_BLK,), lambda i: (i,)),
        ],
        out_specs=pl.BlockSpec((_TC_BLK,), lambda i: (i,)),
        out_shape=jax.ShapeDtypeStruct((_TC_D,), jnp.float32),
    )(P, zb, theta_base)
    return jnp.concatenate([out_tc, out_sc])


# speedup vs baseline: 1.0016x; 1.0016x over previous
import functools

import jax
import jax.numpy as jnp
from jax import lax
from jax.experimental import pallas as pl
from jax.experimental.pallas import tpu as pltpu
from jax.experimental.pallas import tpu_sc as plsc

_D = 65536
_d = 1024
_NC = 2
_NS = 16
_NW = _NC * _NS

_SC_D = 65536
_TC_D = _D - _SC_D
_RW = _SC_D // _NW
_CH = 32
_NCH = _RW // _CH
_CHW = _CH * _d
_JU = 8

_TC_BLK = 4096


def _sc_body(z_hbm, p_hbm, t_hbm, out_hbm, z_v, pa_v, pb_v, o_v, t_v, s_v, sem_a, sem_b):
    wid = lax.axis_index("s") * _NC + lax.axis_index("c")
    out0 = wid * _RW
    row0 = _TC_D + out0

    pltpu.sync_copy(z_hbm, z_v)
    pltpu.sync_copy(t_hbm.at[pl.ds(row0, _RW)], t_v)

    pltpu.async_copy(p_hbm.at[pl.ds(row0, _CH), :], pa_v, sem_a)
    pltpu.async_copy(p_hbm.at[pl.ds(row0 + _CH, _CH), :], pb_v, sem_b)

    def wait_chunk(buf, sem):
        pltpu.make_async_copy(p_hbm.at[pl.ds(0, _CH), :], buf, sem).wait()

    def compute_chunk(p_v, g):
        for gi in range(_CH // 16):

            @plsc.parallel_loop(
                0, _d // 16, unroll=_JU,
                carry=tuple(jnp.zeros((16,), jnp.float32) for _ in range(16)),
            )
            def accs(j, accs):
                accs = list(accs)
                zj = z_v[pl.ds(j * 16, 16)]
                for r in range(16):
                    pv = p_v[gi * 16 + r, pl.ds(j * 16, 16)]
                    accs[r] = accs[r] + pv * zj
                return tuple(accs)

            off = g * _CH + gi * 16
            for r in range(16):
                base = 32 * r
                a = accs[r]
                s_v[pl.ds(base, 16)] = a
                a = a + s_v[pl.ds(base + 8, 16)]
                s_v[pl.ds(base, 16)] = a
                a = a + s_v[pl.ds(base + 4, 16)]
                s_v[pl.ds(base, 16)] = a
                a = a + s_v[pl.ds(base + 2, 16)]
                s_v[pl.ds(base, 16)] = a
                a = a + s_v[pl.ds(base + 1, 16)]
                o_v[pl.ds(off + r, 16)] = a

    def pair_body(k, _):
        g_a = 2 * k
        wait_chunk(pa_v, sem_a)
        compute_chunk(pa_v, g_a)

        @pl.when(g_a + 2 < _NCH)
        def _():
            pltpu.async_copy(
                p_hbm.at[pl.ds(row0 + (g_a + 2) * _CH, _CH), :], pa_v, sem_a)

        wait_chunk(pb_v, sem_b)
        compute_chunk(pb_v, g_a + 1)

        @pl.when(g_a + 3 < _NCH)
        def _():
            pltpu.async_copy(
                p_hbm.at[pl.ds(row0 + (g_a + 3) * _CH, _CH), :], pb_v, sem_b)

        return 0

    lax.fori_loop(0, _NCH // 2, pair_body, 0)

    def theta_body(k, _):
        sl = pl.ds(k * 16, 16)
        o_v[sl] = o_v[sl] + t_v[sl]
        return 0

    lax.fori_loop(0, _RW // 16, theta_body, 0)

    pltpu.sync_copy(o_v.at[pl.ds(0, _RW)], out_hbm.at[pl.ds(out0, _RW)])


_sc_call = functools.partial(
    pl.kernel,
    out_type=jax.ShapeDtypeStruct((_SC_D,), jnp.float32),
    mesh=plsc.VectorSubcoreMesh(core_axis_name="c", subcore_axis_name="s"),
    scratch_types=[
        pltpu.VMEM((_d,), jnp.float32),
        pltpu.VMEM((_CH, _d), jnp.float32),
        pltpu.VMEM((_CH, _d), jnp.float32),
        pltpu.VMEM((_RW + 16,), jnp.float32),
        pltpu.VMEM((_RW,), jnp.float32),
        pltpu.VMEM((512,), jnp.float32),
        pltpu.SemaphoreType.DMA,
        pltpu.SemaphoreType.DMA,
    ],
)(_sc_body)


def _tc_body(p_ref, z_ref, t_ref, o_ref):
    o_ref[...] = t_ref[...] + jnp.sum(p_ref[...] * z_ref[...], axis=1)


def kernel(z, P, theta_base):
    if _TC_D == 0:
        return _sc_call(z, P, theta_base)
    zb = z.reshape(1, _d)
    out_sc = _sc_call(z, P, theta_base)
    out_tc = pl.pallas_call(
        _tc_body,
        grid=(_TC_D // _TC_BLK,),
        in_specs=[
            pl.BlockSpec((_TC_BLK, _d), lambda i: (i, 0)),
            pl.BlockSpec((1, _d), lambda i: (0, 0)),
            pl.BlockSpec((_TC_BLK,), lambda i: (i,)),
        ],
        out_specs=pl.BlockSpec((_TC_BLK,), lambda i: (i,)),
        out_shape=jax.ShapeDtypeStruct((_TC_D,), jnp.float32),
    )(P, zb, theta_base)
    return jnp.concatenate([out_tc, out_sc])


# speedup vs baseline: 1.0942x; 1.0924x over previous
import functools

import jax
import jax.numpy as jnp
from jax import lax
from jax.experimental import pallas as pl
from jax.experimental.pallas import tpu as pltpu
from jax.experimental.pallas import tpu_sc as plsc

_D = 65536
_d = 1024
_NC = 2
_NS = 16
_NW = _NC * _NS

_SC_D = 65536
_TC_D = _D - _SC_D
_RW = _SC_D // _NW
_CH = 32
_NCH = _RW // _CH
_CHW = _CH * _d
_JU = 8

_TC_BLK = 4096


def _sc_body(z_hbm, p_hbm, t_hbm, out_hbm, z_v, pa_v, pb_v, o_v, t_v, s_v, sem_a, sem_b):
    wid = lax.axis_index("s") * _NC + lax.axis_index("c")
    out0 = wid * _RW
    row0 = _TC_D + out0

    pltpu.sync_copy(z_hbm, z_v)
    pltpu.sync_copy(t_hbm.at[pl.ds(row0, _RW)], t_v)

    pltpu.async_copy(p_hbm.at[pl.ds(row0, _CH), :], pa_v, sem_a)
    pltpu.async_copy(p_hbm.at[pl.ds(row0 + _CH, _CH), :], pb_v, sem_b)

    def wait_chunk(buf, sem):
        pltpu.make_async_copy(p_hbm.at[pl.ds(0, _CH), :], buf, sem).wait()

    def compute_chunk(p_v, g):
        for gi in range(_CH // 16):

            @plsc.parallel_loop(
                0, _d // 16, unroll=_JU,
                carry=tuple(jnp.zeros((16,), jnp.float32) for _ in range(16)),
            )
            def accs(j, accs):
                accs = list(accs)
                zj = z_v[pl.ds(j * 16, 16)]
                for r in range(16):
                    pv = p_v[gi * 16 + r, pl.ds(j * 16, 16)]
                    accs[r] = accs[r] + pv * zj
                return tuple(accs)

            off = g * _CH + gi * 16
            for r in range(16):
                s_v[pl.ds(32 * r, 16)] = accs[r]

            @plsc.parallel_loop(0, 16, unroll=16)
            def _(r):
                base = r * 32
                a = s_v[pl.ds(base, 16)] + s_v[pl.ds(base + 8, 16)]
                s_v[pl.ds(base, 16)] = a
                a = a + s_v[pl.ds(base + 4, 16)]
                s_v[pl.ds(base, 16)] = a
                a = a + s_v[pl.ds(base + 2, 16)]
                s_v[pl.ds(base, 16)] = a
                a = a + s_v[pl.ds(base + 1, 16)]
                s_v[pl.ds(base, 16)] = a

            for r in range(16):
                o_v[pl.ds(off + r, 16)] = s_v[pl.ds(32 * r, 16)]

    def pair_body(k, _):
        g_a = 2 * k
        wait_chunk(pa_v, sem_a)
        compute_chunk(pa_v, g_a)

        @pl.when(g_a + 2 < _NCH)
        def _():
            pltpu.async_copy(
                p_hbm.at[pl.ds(row0 + (g_a + 2) * _CH, _CH), :], pa_v, sem_a)

        wait_chunk(pb_v, sem_b)
        compute_chunk(pb_v, g_a + 1)

        @pl.when(g_a + 3 < _NCH)
        def _():
            pltpu.async_copy(
                p_hbm.at[pl.ds(row0 + (g_a + 3) * _CH, _CH), :], pb_v, sem_b)

        return 0

    lax.fori_loop(0, _NCH // 2, pair_body, 0)

    def theta_body(k, _):
        sl = pl.ds(k * 16, 16)
        o_v[sl] = o_v[sl] + t_v[sl]
        return 0

    lax.fori_loop(0, _RW // 16, theta_body, 0)

    pltpu.sync_copy(o_v.at[pl.ds(0, _RW)], out_hbm.at[pl.ds(out0, _RW)])


_sc_call = functools.partial(
    pl.kernel,
    out_type=jax.ShapeDtypeStruct((_SC_D,), jnp.float32),
    mesh=plsc.VectorSubcoreMesh(core_axis_name="c", subcore_axis_name="s"),
    scratch_types=[
        pltpu.VMEM((_d,), jnp.float32),
        pltpu.VMEM((_CH, _d), jnp.float32),
        pltpu.VMEM((_CH, _d), jnp.float32),
        pltpu.VMEM((_RW + 16,), jnp.float32),
        pltpu.VMEM((_RW,), jnp.float32),
        pltpu.VMEM((512,), jnp.float32),
        pltpu.SemaphoreType.DMA,
        pltpu.SemaphoreType.DMA,
    ],
)(_sc_body)


def _tc_body(p_ref, z_ref, t_ref, o_ref):
    o_ref[...] = t_ref[...] + jnp.sum(p_ref[...] * z_ref[...], axis=1)


def kernel(z, P, theta_base):
    if _TC_D == 0:
        return _sc_call(z, P, theta_base)
    zb = z.reshape(1, _d)
    out_sc = _sc_call(z, P, theta_base)
    out_tc = pl.pallas_call(
        _tc_body,
        grid=(_TC_D // _TC_BLK,),
        in_specs=[
            pl.BlockSpec((_TC_BLK, _d), lambda i: (i, 0)),
            pl.BlockSpec((1, _d), lambda i: (0, 0)),
            pl.BlockSpec((_TC_BLK,), lambda i: (i,)),
        ],
        out_specs=pl.BlockSpec((_TC_BLK,), lambda i: (i,)),
        out_shape=jax.ShapeDtypeStruct((_TC_D,), jnp.float32),
    )(P, zb, theta_base)
    return jnp.concatenate([out_tc, out_sc])


# speedup vs baseline: 1.4688x; 1.3423x over previous
import functools

import jax
import jax.numpy as jnp
from jax import lax
from jax.experimental import pallas as pl
from jax.experimental.pallas import tpu as pltpu
from jax.experimental.pallas import tpu_sc as plsc

_D = 65536
_d = 1024
_NC = 2
_NS = 16
_NW = _NC * _NS

_SC_D = 16384
_TC_D = _D - _SC_D
_RW = _SC_D // _NW
_CH = 32
_NCH = _RW // _CH
_CHW = _CH * _d
_JU = 8

_TC_BLK = 2048


def _sc_body(z_hbm, p_hbm, t_hbm, out_hbm, z_v, pa_v, pb_v, o_v, t_v, s_v, sem_a, sem_b):
    wid = lax.axis_index("s") * _NC + lax.axis_index("c")
    out0 = wid * _RW
    row0 = _TC_D + out0

    pltpu.sync_copy(z_hbm, z_v)
    pltpu.sync_copy(t_hbm.at[pl.ds(row0, _RW)], t_v)

    pltpu.async_copy(p_hbm.at[pl.ds(row0, _CH), :], pa_v, sem_a)
    pltpu.async_copy(p_hbm.at[pl.ds(row0 + _CH, _CH), :], pb_v, sem_b)

    def wait_chunk(buf, sem):
        pltpu.make_async_copy(p_hbm.at[pl.ds(0, _CH), :], buf, sem).wait()

    def compute_chunk(p_v, g):
        for gi in range(_CH // 16):

            @plsc.parallel_loop(
                0, _d // 16, unroll=_JU,
                carry=tuple(jnp.zeros((16,), jnp.float32) for _ in range(16)),
            )
            def accs(j, accs):
                accs = list(accs)
                zj = z_v[pl.ds(j * 16, 16)]
                for r in range(16):
                    pv = p_v[gi * 16 + r, pl.ds(j * 16, 16)]
                    accs[r] = accs[r] + pv * zj
                return tuple(accs)

            off = g * _CH + gi * 16
            for r in range(16):
                s_v[pl.ds(32 * r, 16)] = accs[r]

            @plsc.parallel_loop(0, 16, unroll=16)
            def _(r):
                base = r * 32
                a = s_v[pl.ds(base, 16)] + s_v[pl.ds(base + 8, 16)]
                s_v[pl.ds(base, 16)] = a
                a = a + s_v[pl.ds(base + 4, 16)]
                s_v[pl.ds(base, 16)] = a
                a = a + s_v[pl.ds(base + 2, 16)]
                s_v[pl.ds(base, 16)] = a
                a = a + s_v[pl.ds(base + 1, 16)]
                s_v[pl.ds(base, 16)] = a

            for r in range(16):
                o_v[pl.ds(off + r, 16)] = s_v[pl.ds(32 * r, 16)]

    def pair_body(k, _):
        g_a = 2 * k
        wait_chunk(pa_v, sem_a)
        compute_chunk(pa_v, g_a)

        @pl.when(g_a + 2 < _NCH)
        def _():
            pltpu.async_copy(
                p_hbm.at[pl.ds(row0 + (g_a + 2) * _CH, _CH), :], pa_v, sem_a)

        wait_chunk(pb_v, sem_b)
        compute_chunk(pb_v, g_a + 1)

        @pl.when(g_a + 3 < _NCH)
        def _():
            pltpu.async_copy(
                p_hbm.at[pl.ds(row0 + (g_a + 3) * _CH, _CH), :], pb_v, sem_b)

        return 0

    lax.fori_loop(0, _NCH // 2, pair_body, 0)

    def theta_body(k, _):
        sl = pl.ds(k * 16, 16)
        o_v[sl] = o_v[sl] + t_v[sl]
        return 0

    lax.fori_loop(0, _RW // 16, theta_body, 0)

    pltpu.sync_copy(o_v.at[pl.ds(0, _RW)], out_hbm.at[pl.ds(out0, _RW)])


_sc_call = functools.partial(
    pl.kernel,
    out_type=jax.ShapeDtypeStruct((_SC_D,), jnp.float32),
    mesh=plsc.VectorSubcoreMesh(core_axis_name="c", subcore_axis_name="s"),
    scratch_types=[
        pltpu.VMEM((_d,), jnp.float32),
        pltpu.VMEM((_CH, _d), jnp.float32),
        pltpu.VMEM((_CH, _d), jnp.float32),
        pltpu.VMEM((_RW + 16,), jnp.float32),
        pltpu.VMEM((_RW,), jnp.float32),
        pltpu.VMEM((512,), jnp.float32),
        pltpu.SemaphoreType.DMA,
        pltpu.SemaphoreType.DMA,
    ],
)(_sc_body)


def _tc_body(p_ref, z_ref, t_ref, o_ref):
    o_ref[...] = t_ref[...] + jnp.sum(p_ref[...] * z_ref[...], axis=1)


def kernel(z, P, theta_base):
    if _TC_D == 0:
        return _sc_call(z, P, theta_base)
    zb = z.reshape(1, _d)
    out_sc = _sc_call(z, P, theta_base)
    out_tc = pl.pallas_call(
        _tc_body,
        grid=(_TC_D // _TC_BLK,),
        in_specs=[
            pl.BlockSpec((_TC_BLK, _d), lambda i: (i, 0)),
            pl.BlockSpec((1, _d), lambda i: (0, 0)),
            pl.BlockSpec((_TC_BLK,), lambda i: (i,)),
        ],
        out_specs=pl.BlockSpec((_TC_BLK,), lambda i: (i,)),
        out_shape=jax.ShapeDtypeStruct((_TC_D,), jnp.float32),
    )(P, zb, theta_base)
    return jnp.concatenate([out_tc, out_sc])


# speedup vs baseline: 1.4776x; 1.0060x over previous
import functools

import jax
import jax.numpy as jnp
from jax import lax
from jax.experimental import pallas as pl
from jax.experimental.pallas import tpu as pltpu
from jax.experimental.pallas import tpu_sc as plsc

_D = 65536
_d = 1024
_NC = 2
_NS = 16
_NW = _NC * _NS

_SC_D = 8192
_TC_D = _D - _SC_D
_RW = _SC_D // _NW
_CH = 32
_NCH = _RW // _CH
_CHW = _CH * _d
_JU = 8

_TC_BLK = 2048


def _sc_body(z_hbm, p_hbm, t_hbm, out_hbm, z_v, pa_v, pb_v, o_v, t_v, s_v, sem_a, sem_b):
    wid = lax.axis_index("s") * _NC + lax.axis_index("c")
    out0 = wid * _RW
    row0 = _TC_D + out0

    pltpu.sync_copy(z_hbm, z_v)
    pltpu.sync_copy(t_hbm.at[pl.ds(row0, _RW)], t_v)

    pltpu.async_copy(p_hbm.at[pl.ds(row0, _CH), :], pa_v, sem_a)
    pltpu.async_copy(p_hbm.at[pl.ds(row0 + _CH, _CH), :], pb_v, sem_b)

    def wait_chunk(buf, sem):
        pltpu.make_async_copy(p_hbm.at[pl.ds(0, _CH), :], buf, sem).wait()

    def compute_chunk(p_v, g):
        for gi in range(_CH // 16):

            @plsc.parallel_loop(
                0, _d // 16, unroll=_JU,
                carry=tuple(jnp.zeros((16,), jnp.float32) for _ in range(16)),
            )
            def accs(j, accs):
                accs = list(accs)
                zj = z_v[pl.ds(j * 16, 16)]
                for r in range(16):
                    pv = p_v[gi * 16 + r, pl.ds(j * 16, 16)]
                    accs[r] = accs[r] + pv * zj
                return tuple(accs)

            off = g * _CH + gi * 16
            for r in range(16):
                s_v[pl.ds(32 * r, 16)] = accs[r]

            @plsc.parallel_loop(0, 16, unroll=16)
            def _(r):
                base = r * 32
                a = s_v[pl.ds(base, 16)] + s_v[pl.ds(base + 8, 16)]
                s_v[pl.ds(base, 16)] = a
                a = a + s_v[pl.ds(base + 4, 16)]
                s_v[pl.ds(base, 16)] = a
                a = a + s_v[pl.ds(base + 2, 16)]
                s_v[pl.ds(base, 16)] = a
                a = a + s_v[pl.ds(base + 1, 16)]
                s_v[pl.ds(base, 16)] = a

            for r in range(16):
                o_v[pl.ds(off + r, 16)] = s_v[pl.ds(32 * r, 16)]

    def pair_body(k, _):
        g_a = 2 * k
        wait_chunk(pa_v, sem_a)
        compute_chunk(pa_v, g_a)

        @pl.when(g_a + 2 < _NCH)
        def _():
            pltpu.async_copy(
                p_hbm.at[pl.ds(row0 + (g_a + 2) * _CH, _CH), :], pa_v, sem_a)

        wait_chunk(pb_v, sem_b)
        compute_chunk(pb_v, g_a + 1)

        @pl.when(g_a + 3 < _NCH)
        def _():
            pltpu.async_copy(
                p_hbm.at[pl.ds(row0 + (g_a + 3) * _CH, _CH), :], pb_v, sem_b)

        return 0

    lax.fori_loop(0, _NCH // 2, pair_body, 0)

    def theta_body(k, _):
        sl = pl.ds(k * 16, 16)
        o_v[sl] = o_v[sl] + t_v[sl]
        return 0

    lax.fori_loop(0, _RW // 16, theta_body, 0)

    pltpu.sync_copy(o_v.at[pl.ds(0, _RW)], out_hbm.at[pl.ds(out0, _RW)])


_sc_call = functools.partial(
    pl.kernel,
    out_type=jax.ShapeDtypeStruct((_SC_D,), jnp.float32),
    mesh=plsc.VectorSubcoreMesh(core_axis_name="c", subcore_axis_name="s"),
    scratch_types=[
        pltpu.VMEM((_d,), jnp.float32),
        pltpu.VMEM((_CH, _d), jnp.float32),
        pltpu.VMEM((_CH, _d), jnp.float32),
        pltpu.VMEM((_RW + 16,), jnp.float32),
        pltpu.VMEM((_RW,), jnp.float32),
        pltpu.VMEM((512,), jnp.float32),
        pltpu.SemaphoreType.DMA,
        pltpu.SemaphoreType.DMA,
    ],
)(_sc_body)


def _tc_body(p_ref, z_ref, t_ref, o_ref):
    o_ref[...] = t_ref[...] + jnp.sum(p_ref[...] * z_ref[...], axis=1)


def kernel(z, P, theta_base):
    if _TC_D == 0:
        return _sc_call(z, P, theta_base)
    zb = z.reshape(1, _d)
    out_sc = _sc_call(z, P, theta_base)
    out_tc = pl.pallas_call(
        _tc_body,
        grid=(_TC_D // _TC_BLK,),
        in_specs=[
            pl.BlockSpec((_TC_BLK, _d), lambda i: (i, 0)),
            pl.BlockSpec((1, _d), lambda i: (0, 0)),
            pl.BlockSpec((_TC_BLK,), lambda i: (i,)),
        ],
        out_specs=pl.BlockSpec((_TC_BLK,), lambda i: (i,)),
        out_shape=jax.ShapeDtypeStruct((_TC_D,), jnp.float32),
    )(P, zb, theta_base)
    return jnp.concatenate([out_tc, out_sc])
